# baseline (speedup 1.0000x reference)
import functools

import jax
import jax.numpy as jnp
from jax import lax
from jax.experimental import pallas as pl
from jax.experimental.pallas import tpu as pltpu
from jax.experimental.pallas import tpu_sc as plsc

T = 2048
C = 1024
E = 16
K = 2
F = 256
RB = 128
NB = T * K // RB + E
P = NB * RB

NC = 2
NS = 16
NW = NC * NS
PAIRS = T * K
PER_W = PAIRS // NW
CH = 64
PER_T = T // NW
CH2 = 32


def _shift_down(s, sh):
    return jnp.concatenate([jnp.zeros((sh, s.shape[1]), s.dtype), s[:-sh]], 0)


def _excl_cumsum_rows(a):
    s = a
    sh = 1
    while sh < a.shape[0]:
        s = s + _shift_down(s, sh)
        sh *= 2
    return s - a


def _router_kernel(x_ref, gate_ref, pos0_ref, pos1_ref, pv0_ref, pv1_ref,
                   be_ref):
    r = jax.lax.dot_general(x_ref[...], gate_ref[...],
                            (((1,), (1,)), ((), ())),
                            preferred_element_type=jnp.float32)
    ids = jax.lax.broadcasted_iota(jnp.int32, (T, E), 1)
    m1 = jnp.max(r, axis=1, keepdims=True)
    i1 = jnp.min(jnp.where(r == m1, ids, E), axis=1, keepdims=True)
    masked = jnp.where(ids == i1, -jnp.inf, r)
    m2 = jnp.max(masked, axis=1, keepdims=True)
    i2 = jnp.min(jnp.where(masked == m2, ids, E), axis=1, keepdims=True)
    p1 = 1.0 / (1.0 + jnp.exp(m2 - m1))
    lane0 = jnp.zeros((T, 16), jnp.float32)
    pv0_ref[...] = p1 + lane0
    pv1_ref[...] = (1.0 - p1) + lane0

    oh0 = (ids == i1).astype(jnp.int32)
    oh1 = (ids == i2).astype(jnp.int32)
    c0 = _excl_cumsum_rows(oh0)
    c1 = _excl_cumsum_rows(oh1)
    cnt0 = jnp.sum(oh0, axis=0, keepdims=True)
    cnt = cnt0 + jnp.sum(oh1, axis=0, keepdims=True)
    nblk = (cnt + (RB - 1)) // RB
    ei = jax.lax.broadcasted_iota(jnp.int32, (E, E), 0)
    ej = jax.lax.broadcasted_iota(jnp.int32, (E, E), 1)
    lex = (ei < ej).astype(jnp.float32)
    bs = jax.lax.dot_general(nblk.astype(jnp.float32), lex,
                             (((1,), (0,)), ((), ())),
                             preferred_element_type=jnp.float32)
    bs = bs.astype(jnp.int32)
    po = bs * RB
    rank0 = jnp.sum(c0 * oh0, axis=1, keepdims=True)
    rank1 = jnp.sum((cnt0 + c1) * oh1, axis=1, keepdims=True)
    pos0_ref[...] = jnp.sum(po * oh0, axis=1, keepdims=True) + rank0
    pos1_ref[...] = jnp.sum(po * oh1, axis=1, keepdims=True) + rank1

    bi = jax.lax.broadcasted_iota(jnp.int32, (NB, E), 0)
    be_ref[...] = jnp.sum((bi >= bs).astype(jnp.int32), axis=1,
                          keepdims=True) - 1


def _run_router(xf, gate_w):
    return pl.pallas_call(
        _router_kernel,
        out_shape=(
            jax.ShapeDtypeStruct((T, 1), jnp.int32),
            jax.ShapeDtypeStruct((T, 1), jnp.int32),
            jax.ShapeDtypeStruct((T, 16), jnp.float32),
            jax.ShapeDtypeStruct((T, 16), jnp.float32),
            jax.ShapeDtypeStruct((NB, 1), jnp.int32),
        ),
    )(xf, gate_w)


_sc_mesh = plsc.VectorSubcoreMesh(core_axis_name="c", subcore_axis_name="s")


@functools.partial(
    pl.kernel,
    mesh=_sc_mesh,
    out_type=jax.ShapeDtypeStruct((P, C), jnp.float32),
    scratch_types=[
        pltpu.VMEM((PER_W // CH, CH), jnp.int32),
        pltpu.VMEM((CH, C), jnp.float32),
        pltpu.SemaphoreType.DMA,
    ],
)
def _sc_scatter(pos_hbm, x_hbm, xs_hbm, idx_v, rows_v, sem1):
    wid = lax.axis_index("s") * NC + lax.axis_index("c")
    base = wid * PER_W
    tok_base = base % T
    for c in range(PER_W // CH):
        pltpu.sync_copy(pos_hbm.at[pl.ds(base + c * CH, CH)], idx_v.at[c])
    for c in range(PER_W // CH):
        pltpu.sync_copy(x_hbm.at[pl.ds(tok_base + c * CH, CH)], rows_v)
        pltpu.async_copy(rows_v, xs_hbm.at[idx_v.at[c]], sem1).wait()


def _group_kernel(be_ref, xs_ref, fc1_ref, fc2_ref, proj_ref, o_ref):
    xb = xs_ref[...]
    h = jax.lax.dot_general(xb, fc1_ref[0], (((1,), (1,)), ((), ())),
                            preferred_element_type=jnp.float32)
    g = jax.lax.dot_general(xb, fc2_ref[0], (((1,), (1,)), ((), ())),
                            preferred_element_type=jnp.float32)
    a = (h * jax.lax.logistic(h)) * g
    o_ref[...] = jax.lax.dot_general(a, proj_ref[0], (((1,), (1,)), ((), ())),
                                     preferred_element_type=jnp.float32)


def _run_grouped(be, xs, fc1_w, fc2_w, proj_w):
    grid_spec = pltpu.PrefetchScalarGridSpec(
        num_scalar_prefetch=1,
        grid=(NB,),
        in_specs=[
            pl.BlockSpec((RB, C), lambda b, be: (b, 0)),
            pl.BlockSpec((1, F, C), lambda b, be: (be[b], 0, 0)),
            pl.BlockSpec((1, F, C), lambda b, be: (be[b], 0, 0)),
            pl.BlockSpec((1, C, F), lambda b, be: (be[b], 0, 0)),
        ],
        out_specs=pl.BlockSpec((RB, C), lambda b, be: (b, 0)),
    )
    return pl.pallas_call(
        _group_kernel,
        grid_spec=grid_spec,
        out_shape=jax.ShapeDtypeStruct((P, C), jnp.float32),
    )(be, xs, fc1_w, fc2_w, proj_w)


@functools.partial(
    pl.kernel,
    mesh=_sc_mesh,
    out_type=jax.ShapeDtypeStruct((T, C), jnp.float32),
    scratch_types=[
        pltpu.VMEM((PER_T,), jnp.int32),
        pltpu.VMEM((PER_T,), jnp.int32),
        pltpu.VMEM((PER_T, 16), jnp.float32),
        pltpu.VMEM((PER_T, 16), jnp.float32),
        pltpu.VMEM((CH2, C), jnp.float32),
        pltpu.VMEM((CH2, C), jnp.float32),
        pltpu.SemaphoreType.DMA,
        pltpu.SemaphoreType.DMA,
    ],
)
def _sc_combine(pos0_hbm, pos1_hbm, pv0_hbm, pv1_hbm, os_hbm, y_hbm, i0_v,
                i1_v, p0_v, p1_v, r0_v, r1_v, sem0, sem1):
    wid = lax.axis_index("s") * NC + lax.axis_index("c")
    tbase = wid * PER_T
    pltpu.sync_copy(pos0_hbm.at[pl.ds(tbase, PER_T)], i0_v)
    pltpu.sync_copy(pos1_hbm.at[pl.ds(tbase, PER_T)], i1_v)
    pltpu.sync_copy(pv0_hbm.at[pl.ds(tbase, PER_T)], p0_v)
    pltpu.sync_copy(pv1_hbm.at[pl.ds(tbase, PER_T)], p1_v)
    for c in range(PER_T // CH2):
        cp0 = pltpu.async_copy(os_hbm.at[i0_v.at[pl.ds(c * CH2, CH2)]], r0_v,
                               sem0)
        cp1 = pltpu.async_copy(os_hbm.at[i1_v.at[pl.ds(c * CH2, CH2)]], r1_v,
                               sem1)
        cp0.wait()
        cp1.wait()

        def _row(i, carry):
            p0 = p0_v[c * CH2 + i, :]
            p1 = p1_v[c * CH2 + i, :]
            for j in range(C // 16):
                sl = pl.ds(j * 16, 16)
                r0_v[i, sl] = p0 * r0_v[i, sl] + p1 * r1_v[i, sl]
            return carry

        lax.fori_loop(0, CH2, _row, 0)
        pltpu.sync_copy(r0_v, y_hbm.at[pl.ds(tbase + c * CH2, CH2)])


@jax.jit
def kernel(x, gate_w, fc1_w, fc2_w, proj_w):
    B, Tq, Cq = x.shape
    xf = x.reshape(T, C)
    pos0, pos1, pv0, pv1, be = _run_router(xf, gate_w)
    pos_cat = jnp.concatenate([pos0, pos1], axis=0).reshape(PAIRS)
    xs = _sc_scatter(pos_cat, xf)
    out_sorted = _run_grouped(be.reshape(NB), xs, fc1_w, fc2_w, proj_w)
    y = _sc_combine(pos0.reshape(T), pos1.reshape(T), pv0, pv1, out_sorted)
    return y.reshape(B, Tq, Cq)


# speedup vs baseline: 1.0812x; 1.0812x over previous
import functools

import jax
import jax.numpy as jnp
from jax import lax
from jax.experimental import pallas as pl
from jax.experimental.pallas import tpu as pltpu
from jax.experimental.pallas import tpu_sc as plsc

T = 2048
C = 1024
E = 16
K = 2
F = 256
RB = 128
NB = T * K // RB + E
P = NB * RB

NC = 2
NS = 16
NW = NC * NS
PAIRS = T * K
PER_W = PAIRS // NW
CH = 32
NCH = PER_W // CH
PER_T = T // NW
CH2 = 16
NCH2 = PER_T // CH2


def _shift_down(s, sh):
    return jnp.concatenate([jnp.zeros((sh, s.shape[1]), s.dtype), s[:-sh]], 0)


def _excl_cumsum_rows(a):
    s = a
    sh = 1
    while sh < a.shape[0]:
        s = s + _shift_down(s, sh)
        sh *= 2
    return s - a


def _router_kernel(x_ref, gate_ref, pos_ref, pv0_ref, pv1_ref, be_ref,
                   ub_ref):
    r = jax.lax.dot_general(x_ref[...], gate_ref[...],
                            (((1,), (1,)), ((), ())),
                            preferred_element_type=jnp.float32)
    ids = jax.lax.broadcasted_iota(jnp.int32, (T, E), 1)
    m1 = jnp.max(r, axis=1, keepdims=True)
    i1 = jnp.min(jnp.where(r == m1, ids, E), axis=1, keepdims=True)
    masked = jnp.where(ids == i1, -jnp.inf, r)
    m2 = jnp.max(masked, axis=1, keepdims=True)
    i2 = jnp.min(jnp.where(masked == m2, ids, E), axis=1, keepdims=True)
    p1 = 1.0 / (1.0 + jnp.exp(m2 - m1))
    lane0 = jnp.zeros((T, 16), jnp.float32)
    pv0_ref[...] = p1 + lane0
    pv1_ref[...] = (1.0 - p1) + lane0

    oh0 = (ids == i1).astype(jnp.int32)
    oh1 = (ids == i2).astype(jnp.int32)
    c0 = _excl_cumsum_rows(oh0)
    c1 = _excl_cumsum_rows(oh1)
    cnt0 = jnp.sum(oh0, axis=0, keepdims=True)
    cnt = cnt0 + jnp.sum(oh1, axis=0, keepdims=True)
    nblk = (cnt + (RB - 1)) // RB
    ei = jax.lax.broadcasted_iota(jnp.int32, (E, E), 0)
    ej = jax.lax.broadcasted_iota(jnp.int32, (E, E), 1)
    lex = (ei < ej).astype(jnp.float32)
    bs = jax.lax.dot_general(nblk.astype(jnp.float32), lex,
                             (((1,), (0,)), ((), ())),
                             preferred_element_type=jnp.float32)
    bs = bs.astype(jnp.int32)
    po = bs * RB
    rank0 = jnp.sum(c0 * oh0, axis=1, keepdims=True)
    rank1 = jnp.sum((cnt0 + c1) * oh1, axis=1, keepdims=True)
    pos_ref[0:T] = jnp.sum(po * oh0, axis=1, keepdims=True) + rank0
    pos_ref[T:2 * T] = jnp.sum(po * oh1, axis=1, keepdims=True) + rank1

    bi = jax.lax.broadcasted_iota(jnp.int32, (NB, E), 0)
    be_ref[...] = jnp.sum((bi >= bs).astype(jnp.int32), axis=1,
                          keepdims=True) - 1
    ub_ref[...] = jnp.sum(nblk, axis=1, keepdims=True)


def _run_router(xf, gate_w):
    return pl.pallas_call(
        _router_kernel,
        out_shape=(
            jax.ShapeDtypeStruct((PAIRS, 1), jnp.int32),
            jax.ShapeDtypeStruct((T, 16), jnp.float32),
            jax.ShapeDtypeStruct((T, 16), jnp.float32),
            jax.ShapeDtypeStruct((NB, 1), jnp.int32),
            jax.ShapeDtypeStruct((1, 1), jnp.int32),
        ),
    )(xf, gate_w)


_sc_mesh = plsc.VectorSubcoreMesh(core_axis_name="c", subcore_axis_name="s")


@functools.partial(
    pl.kernel,
    mesh=_sc_mesh,
    out_type=jax.ShapeDtypeStruct((P, C), jnp.float32),
    scratch_types=[
        pltpu.VMEM((NCH, CH), jnp.int32),
        pltpu.VMEM((CH, C), jnp.float32),
        pltpu.VMEM((CH, C), jnp.float32),
        pltpu.SemaphoreType.DMA,
        pltpu.SemaphoreType.DMA,
        pltpu.SemaphoreType.DMA,
        pltpu.SemaphoreType.DMA,
    ],
)
def _sc_scatter(pos_hbm, x_hbm, xs_hbm, idx_v, rows_a, rows_b, semr_a, semr_b,
                sems_a, sems_b):
    wid = lax.axis_index("s") * NC + lax.axis_index("c")
    base = wid * PER_W
    tok_base = base % T
    rows = (rows_a, rows_b)
    semr = (semr_a, semr_b)
    sems = (sems_a, sems_b)
    for c in range(NCH):
        pltpu.sync_copy(pos_hbm.at[pl.ds(base + c * CH, CH)], idx_v.at[c])
    reads = [None] * NCH
    scats = [None] * NCH
    reads[0] = pltpu.async_copy(x_hbm.at[pl.ds(tok_base, CH)], rows[0],
                                semr[0])
    for c in range(NCH):
        p = c % 2
        if c + 1 < NCH:
            if c >= 1:
                scats[c - 1].wait()
            reads[c + 1] = pltpu.async_copy(
                x_hbm.at[pl.ds(tok_base + (c + 1) * CH, CH)],
                rows[(c + 1) % 2], semr[(c + 1) % 2])
        reads[c].wait()
        scats[c] = pltpu.async_copy(rows[p], xs_hbm.at[idx_v.at[c]], sems[p])
    scats[NCH - 2].wait()
    scats[NCH - 1].wait()


def _group_kernel(be_ref, ub_ref, xs_ref, fc1_ref, fc2_ref, proj_ref, o_ref):
    b = pl.program_id(0)

    @pl.when(b < ub_ref[0])
    def _():
        xb = xs_ref[...]
        h = jax.lax.dot_general(xb, fc1_ref[0], (((1,), (1,)), ((), ())),
                                preferred_element_type=jnp.float32)
        g = jax.lax.dot_general(xb, fc2_ref[0], (((1,), (1,)), ((), ())),
                                preferred_element_type=jnp.float32)
        a = (h * jax.lax.logistic(h)) * g
        o_ref[...] = jax.lax.dot_general(a, proj_ref[0],
                                         (((1,), (1,)), ((), ())),
                                         preferred_element_type=jnp.float32)


def _run_grouped(be, ub, xs, fc1_w, fc2_w, proj_w):
    grid_spec = pltpu.PrefetchScalarGridSpec(
        num_scalar_prefetch=2,
        grid=(NB,),
        in_specs=[
            pl.BlockSpec((RB, C),
                         lambda b, be, ub: (jnp.where(b < ub[0], b, 0), 0)),
            pl.BlockSpec((1, F, C), lambda b, be, ub: (be[b], 0, 0)),
            pl.BlockSpec((1, F, C), lambda b, be, ub: (be[b], 0, 0)),
            pl.BlockSpec((1, C, F), lambda b, be, ub: (be[b], 0, 0)),
        ],
        out_specs=pl.BlockSpec(
            (RB, C), lambda b, be, ub: (jnp.where(b < ub[0], b, NB - 1), 0)),
    )
    return pl.pallas_call(
        _group_kernel,
        grid_spec=grid_spec,
        out_shape=jax.ShapeDtypeStruct((P, C), jnp.float32),
    )(be, ub, xs, fc1_w, fc2_w, proj_w)


@functools.partial(
    pl.kernel,
    mesh=_sc_mesh,
    out_type=jax.ShapeDtypeStruct((T, C), jnp.float32),
    scratch_types=[
        pltpu.VMEM((PER_T,), jnp.int32),
        pltpu.VMEM((PER_T,), jnp.int32),
        pltpu.VMEM((PER_T, 16), jnp.float32),
        pltpu.VMEM((PER_T, 16), jnp.float32),
        pltpu.VMEM((CH2, C), jnp.float32),
        pltpu.VMEM((CH2, C), jnp.float32),
        pltpu.VMEM((CH2, C), jnp.float32),
        pltpu.VMEM((CH2, C), jnp.float32),
        pltpu.SemaphoreType.DMA,
        pltpu.SemaphoreType.DMA,
        pltpu.SemaphoreType.DMA,
        pltpu.SemaphoreType.DMA,
        pltpu.SemaphoreType.DMA,
        pltpu.SemaphoreType.DMA,
    ],
)
def _sc_combine(pos_hbm, pv0_hbm, pv1_hbm, os_hbm, y_hbm, i0_v, i1_v, p0_v,
                p1_v, r0_a, r0_b, r1_a, r1_b, sg0_a, sg0_b, sg1_a, sg1_b,
                st_a, st_b):
    wid = lax.axis_index("s") * NC + lax.axis_index("c")
    tbase = wid * PER_T
    pltpu.sync_copy(pos_hbm.at[pl.ds(tbase, PER_T)], i0_v)
    pltpu.sync_copy(pos_hbm.at[pl.ds(T + tbase, PER_T)], i1_v)
    pltpu.sync_copy(pv0_hbm.at[pl.ds(tbase, PER_T)], p0_v)
    pltpu.sync_copy(pv1_hbm.at[pl.ds(tbase, PER_T)], p1_v)
    r0 = (r0_a, r0_b)
    r1 = (r1_a, r1_b)
    sg0 = (sg0_a, sg0_b)
    sg1 = (sg1_a, sg1_b)
    st = (st_a, st_b)

    def gathers(c):
        p = c % 2
        g0 = pltpu.async_copy(os_hbm.at[i0_v.at[pl.ds(c * CH2, CH2)]], r0[p],
                              sg0[p])
        g1 = pltpu.async_copy(os_hbm.at[i1_v.at[pl.ds(c * CH2, CH2)]], r1[p],
                              sg1[p])
        return g0, g1

    pend = [None] * NCH2
    stores = [None] * NCH2
    pend[0] = gathers(0)
    for c in range(NCH2):
        p = c % 2
        if c + 1 < NCH2:
            if c >= 1:
                stores[c - 1].wait()
            pend[c + 1] = gathers(c + 1)
        pend[c][0].wait()
        pend[c][1].wait()

        def _row(i, carry):
            p0 = p0_v[c * CH2 + i, :]
            p1 = p1_v[c * CH2 + i, :]
            for j in range(C // 16):
                sl = pl.ds(j * 16, 16)
                r0[p][i, sl] = p0 * r0[p][i, sl] + p1 * r1[p][i, sl]
            return carry

        lax.fori_loop(0, CH2, _row, 0)
        stores[c] = pltpu.async_copy(r0[p],
                                     y_hbm.at[pl.ds(tbase + c * CH2, CH2)],
                                     st[p])
    stores[NCH2 - 2].wait()
    stores[NCH2 - 1].wait()


@jax.jit
def kernel(x, gate_w, fc1_w, fc2_w, proj_w):
    B, Tq, Cq = x.shape
    xf = x.reshape(T, C)
    pos, pv0, pv1, be, ub = _run_router(xf, gate_w)
    pos_flat = pos.reshape(PAIRS)
    xs = _sc_scatter(pos_flat, xf)
    out_sorted = _run_grouped(be.reshape(NB), ub.reshape(1), xs, fc1_w, fc2_w,
                              proj_w)
    y = _sc_combine(pos_flat, pv0, pv1, out_sorted)
    return y.reshape(B, Tq, Cq)


# speedup vs baseline: 1.1228x; 1.0386x over previous
import functools

import jax
import jax.numpy as jnp
from jax import lax
from jax.experimental import pallas as pl
from jax.experimental.pallas import tpu as pltpu
from jax.experimental.pallas import tpu_sc as plsc

T = 2048
C = 1024
E = 16
K = 2
F = 256
RB = 128
NB = T * K // RB + E
P = NB * RB

NC = 2
NS = 16
NW = NC * NS
PAIRS = T * K
PER_T = T // NW
CH = 32
NCH = PER_T // CH
CH2 = 16
NCH2 = PER_T // CH2


def _shift_down(s, sh):
    return jnp.concatenate([jnp.zeros((sh, s.shape[1]), s.dtype), s[:-sh]], 0)


def _excl_cumsum_rows(a):
    s = a
    sh = 1
    while sh < a.shape[0]:
        s = s + _shift_down(s, sh)
        sh *= 2
    return s - a


def _router_kernel(x_ref, gate_ref, pos_ref, pv0_ref, pv1_ref, be_ref,
                   ub_ref):
    r = jax.lax.dot_general(x_ref[...], gate_ref[...],
                            (((1,), (1,)), ((), ())),
                            preferred_element_type=jnp.float32)
    ids = jax.lax.broadcasted_iota(jnp.int32, (T, E), 1)
    m1 = jnp.max(r, axis=1, keepdims=True)
    i1 = jnp.min(jnp.where(r == m1, ids, E), axis=1, keepdims=True)
    masked = jnp.where(ids == i1, -jnp.inf, r)
    m2 = jnp.max(masked, axis=1, keepdims=True)
    i2 = jnp.min(jnp.where(masked == m2, ids, E), axis=1, keepdims=True)
    p1 = 1.0 / (1.0 + jnp.exp(m2 - m1))
    lane0 = jnp.zeros((T, 128), jnp.float32)
    pv0_ref[...] = p1 + lane0
    pv1_ref[...] = (1.0 - p1) + lane0

    oh0 = (ids == i1).astype(jnp.int32)
    oh1 = (ids == i2).astype(jnp.int32)
    c0 = _excl_cumsum_rows(oh0)
    c1 = _excl_cumsum_rows(oh1)
    cnt0 = jnp.sum(oh0, axis=0, keepdims=True)
    cnt = cnt0 + jnp.sum(oh1, axis=0, keepdims=True)
    nblk = (cnt + (RB - 1)) // RB
    ei = jax.lax.broadcasted_iota(jnp.int32, (E, E), 0)
    ej = jax.lax.broadcasted_iota(jnp.int32, (E, E), 1)
    lex = (ei < ej).astype(jnp.float32)
    bs = jax.lax.dot_general(nblk.astype(jnp.float32), lex,
                             (((1,), (0,)), ((), ())),
                             preferred_element_type=jnp.float32)
    bs = bs.astype(jnp.int32)
    po = bs * RB
    rank0 = jnp.sum(c0 * oh0, axis=1, keepdims=True)
    rank1 = jnp.sum((cnt0 + c1) * oh1, axis=1, keepdims=True)
    pos_ref[0:T] = jnp.sum(po * oh0, axis=1, keepdims=True) + rank0
    pos_ref[T:2 * T] = jnp.sum(po * oh1, axis=1, keepdims=True) + rank1

    bi = jax.lax.broadcasted_iota(jnp.int32, (NB, E), 0)
    be_ref[...] = jnp.sum((bi >= bs).astype(jnp.int32), axis=1,
                          keepdims=True) - 1
    ub_ref[...] = jnp.sum(nblk, axis=1, keepdims=True)


def _run_router(xf, gate_w):
    return pl.pallas_call(
        _router_kernel,
        out_shape=(
            jax.ShapeDtypeStruct((PAIRS, 1), jnp.int32),
            jax.ShapeDtypeStruct((T, 128), jnp.float32),
            jax.ShapeDtypeStruct((T, 128), jnp.float32),
            jax.ShapeDtypeStruct((NB, 1), jnp.int32),
            jax.ShapeDtypeStruct((1, 1), jnp.int32),
        ),
    )(xf, gate_w)


_sc_mesh = plsc.VectorSubcoreMesh(core_axis_name="c", subcore_axis_name="s")


@functools.partial(
    pl.kernel,
    mesh=_sc_mesh,
    out_type=(jax.ShapeDtypeStruct((P, C), jnp.float32),
              jax.ShapeDtypeStruct((P, 128), jnp.float32)),
    scratch_types=[
        pltpu.VMEM((NCH, CH), jnp.int32),
        pltpu.VMEM((NCH, CH), jnp.int32),
        pltpu.VMEM((CH, C), jnp.float32),
        pltpu.VMEM((CH, C), jnp.float32),
        pltpu.VMEM((CH, 128), jnp.float32),
        pltpu.VMEM((CH, 128), jnp.float32),
        pltpu.VMEM((CH, 128), jnp.float32),
        pltpu.VMEM((CH, 128), jnp.float32),
        pltpu.SemaphoreType.DMA,
        pltpu.SemaphoreType.DMA,
        pltpu.SemaphoreType.DMA,
        pltpu.SemaphoreType.DMA,
        pltpu.SemaphoreType.DMA,
        pltpu.SemaphoreType.DMA,
        pltpu.SemaphoreType.DMA,
        pltpu.SemaphoreType.DMA,
        pltpu.SemaphoreType.DMA,
        pltpu.SemaphoreType.DMA,
    ],
)
def _sc_scatter(pos_hbm, pv0_hbm, pv1_hbm, x_hbm, xs_hbm, ps_hbm, i0_v, i1_v,
                rows_a, rows_b, p0_a, p0_b, p1_a, p1_b, semr_a, semr_b,
                sx0_a, sx0_b, sx1_a, sx1_b, sp0_a, sp0_b, sp1_a, sp1_b):
    wid = lax.axis_index("s") * NC + lax.axis_index("c")
    tbase = wid * PER_T
    rows = (rows_a, rows_b)
    p0b = (p0_a, p0_b)
    p1b = (p1_a, p1_b)
    semr = (semr_a, semr_b)
    sx0 = (sx0_a, sx0_b)
    sx1 = (sx1_a, sx1_b)
    sp0 = (sp0_a, sp0_b)
    sp1 = (sp1_a, sp1_b)
    reads = [None] * NCH
    for c in range(NCH):
        reads[c] = pltpu.async_copy(x_hbm.at[pl.ds(tbase + c * CH, CH)],
                                    rows[c], semr[c])
    for c in range(NCH):
        sl = pl.ds(tbase + c * CH, CH)
        pltpu.sync_copy(pos_hbm.at[sl], i0_v.at[c])
        pltpu.sync_copy(pos_hbm.at[pl.ds(T + tbase + c * CH, CH)], i1_v.at[c])
        pltpu.sync_copy(pv0_hbm.at[sl], p0b[c])
        pltpu.sync_copy(pv1_hbm.at[sl], p1b[c])
    scats = []
    for c in range(NCH):
        reads[c].wait()
        scats.append(pltpu.async_copy(rows[c], xs_hbm.at[i0_v.at[c]], sx0[c]))
        scats.append(pltpu.async_copy(rows[c], xs_hbm.at[i1_v.at[c]], sx1[c]))
        scats.append(pltpu.async_copy(p0b[c], ps_hbm.at[i0_v.at[c]], sp0[c]))
        scats.append(pltpu.async_copy(p1b[c], ps_hbm.at[i1_v.at[c]], sp1[c]))
    for s in scats:
        s.wait()


def _group_kernel(be_ref, ub_ref, xs_ref, ps_ref, fc1_ref, fc2_ref, proj_ref,
                  o_ref):
    b = pl.program_id(0)

    @pl.when(b < ub_ref[0])
    def _():
        xb = xs_ref[...]
        h = jax.lax.dot_general(xb, fc1_ref[0], (((1,), (1,)), ((), ())),
                                preferred_element_type=jnp.float32)
        g = jax.lax.dot_general(xb, fc2_ref[0], (((1,), (1,)), ((), ())),
                                preferred_element_type=jnp.float32)
        a = (h * jax.lax.logistic(h)) * g
        o = jax.lax.dot_general(a, proj_ref[0], (((1,), (1,)), ((), ())),
                                preferred_element_type=jnp.float32)
        o_ref[...] = o * ps_ref[:, 0:1]


def _run_grouped(be, ub, xs, ps, fc1_w, fc2_w, proj_w):
    grid_spec = pltpu.PrefetchScalarGridSpec(
        num_scalar_prefetch=2,
        grid=(NB,),
        in_specs=[
            pl.BlockSpec((RB, C),
                         lambda b, be, ub: (jnp.where(b < ub[0], b, 0), 0)),
            pl.BlockSpec((RB, 128),
                         lambda b, be, ub: (jnp.where(b < ub[0], b, 0), 0)),
            pl.BlockSpec((1, F, C), lambda b, be, ub: (be[b], 0, 0)),
            pl.BlockSpec((1, F, C), lambda b, be, ub: (be[b], 0, 0)),
            pl.BlockSpec((1, C, F), lambda b, be, ub: (be[b], 0, 0)),
        ],
        out_specs=pl.BlockSpec(
            (RB, C), lambda b, be, ub: (jnp.where(b < ub[0], b, NB - 1), 0)),
    )
    return pl.pallas_call(
        _group_kernel,
        grid_spec=grid_spec,
        out_shape=jax.ShapeDtypeStruct((P, C), jnp.float32),
    )(be, ub, xs, ps, fc1_w, fc2_w, proj_w)


@functools.partial(
    pl.kernel,
    mesh=_sc_mesh,
    out_type=jax.ShapeDtypeStruct((T, C), jnp.float32),
    scratch_types=[
        pltpu.VMEM((PER_T,), jnp.int32),
        pltpu.VMEM((PER_T,), jnp.int32),
        pltpu.VMEM((CH2, C), jnp.float32),
        pltpu.VMEM((CH2, C), jnp.float32),
        pltpu.VMEM((CH2, C), jnp.float32),
        pltpu.VMEM((CH2, C), jnp.float32),
        pltpu.SemaphoreType.DMA,
        pltpu.SemaphoreType.DMA,
        pltpu.SemaphoreType.DMA,
        pltpu.SemaphoreType.DMA,
        pltpu.SemaphoreType.DMA,
        pltpu.SemaphoreType.DMA,
    ],
)
def _sc_combine(pos_hbm, os_hbm, y_hbm, i0_v, i1_v, r0_a, r0_b, r1_a, r1_b,
                sg0_a, sg0_b, sg1_a, sg1_b, st_a, st_b):
    wid = lax.axis_index("s") * NC + lax.axis_index("c")
    tbase = wid * PER_T
    pltpu.sync_copy(pos_hbm.at[pl.ds(tbase, PER_T)], i0_v)
    pltpu.sync_copy(pos_hbm.at[pl.ds(T + tbase, PER_T)], i1_v)
    r0 = (r0_a, r0_b)
    r1 = (r1_a, r1_b)
    sg0 = (sg0_a, sg0_b)
    sg1 = (sg1_a, sg1_b)
    st = (st_a, st_b)

    def gathers(c):
        p = c % 2
        g0 = pltpu.async_copy(os_hbm.at[i0_v.at[pl.ds(c * CH2, CH2)]], r0[p],
                              sg0[p])
        g1 = pltpu.async_copy(os_hbm.at[i1_v.at[pl.ds(c * CH2, CH2)]], r1[p],
                              sg1[p])
        return g0, g1

    pend = [None] * NCH2
    stores = [None] * NCH2
    pend[0] = gathers(0)
    for c in range(NCH2):
        p = c % 2
        if c + 1 < NCH2:
            if c >= 1:
                stores[c - 1].wait()
            pend[c + 1] = gathers(c + 1)
        pend[c][0].wait()
        pend[c][1].wait()

        def _row(i, carry):
            for j in range(C // 16):
                sl = pl.ds(j * 16, 16)
                r0[p][i, sl] = r0[p][i, sl] + r1[p][i, sl]
            return carry

        lax.fori_loop(0, CH2, _row, 0)
        stores[c] = pltpu.async_copy(r0[p],
                                     y_hbm.at[pl.ds(tbase + c * CH2, CH2)],
                                     st[p])
    stores[NCH2 - 2].wait()
    stores[NCH2 - 1].wait()


@jax.jit
def kernel(x, gate_w, fc1_w, fc2_w, proj_w):
    B, Tq, Cq = x.shape
    xf = x.reshape(T, C)
    pos, pv0, pv1, be, ub = _run_router(xf, gate_w)
    pos_flat = pos.reshape(PAIRS)
    xs, ps = _sc_scatter(pos_flat, pv0, pv1, xf)
    out_sorted = _run_grouped(be.reshape(NB), ub.reshape(1), xs, ps, fc1_w,
                              fc2_w, proj_w)
    y = _sc_combine(pos_flat, out_sorted)
    return y.reshape(B, Tq, Cq)
